# Initial kernel scaffold; baseline (speedup 1.0000x reference)
import jax
import jax.numpy as jnp
from jax.experimental import pallas as pl


def kernel(x, edge_index, W1, b1, W2, b2):
    raise NotImplementedError("write your pallas kernel here")


# speedup vs baseline: 32.5782x; 32.5782x over previous
import functools

import jax
import jax.numpy as jnp
from jax import lax
from jax.experimental import pallas as pl
from jax.experimental.pallas import tpu as pltpu
from jax.experimental.pallas import tpu_sc as plsc

NC = 2
NS = 16
NW = NC * NS
L = 16

_MESH = plsc.VectorSubcoreMesh(core_axis_name="c", subcore_axis_name="s")
_SC_PARAMS = pltpu.CompilerParams(needs_layout_passes=False)


def _degree(dst, n_nodes):
    e = dst.shape[0]
    epw = e // NW

    @functools.partial(
        pl.kernel,
        out_type=jax.ShapeDtypeStruct((NW, n_nodes), jnp.float32),
        mesh=_MESH,
        compiler_params=_SC_PARAMS,
        scratch_types=[
            pltpu.VMEM((n_nodes,), jnp.float32),
            pltpu.VMEM((epw,), jnp.int32),
        ],
    )
    def deg_kernel(dst_hbm, out_hbm, cnt_v, idx_v):
        c = lax.axis_index("c")
        s = lax.axis_index("s")
        wid = c * NS + s
        pltpu.sync_copy(dst_hbm.at[pl.ds(wid * epw, epw)], idx_v)

        def zero_body(i, _):
            cnt_v[pl.ds(i * L, L)] = jnp.zeros((L,), jnp.float32)
            return 0

        lax.fori_loop(0, n_nodes // L, zero_body, 0)

        ones = jnp.full((L,), 1.0, jnp.float32)

        def body(i, _):
            idx = idx_v[pl.ds(i * L, L)]
            plsc.addupdate_scatter(cnt_v, [idx], ones)
            return 0

        lax.fori_loop(0, epw // L, body, 0)
        pltpu.sync_copy(cnt_v, out_hbm.at[wid])

    return deg_kernel(dst)


def _mm1_body(x_ref, w_ref, cnt_ref, h_ref, g_ref):
    h = jnp.dot(x_ref[...], w_ref[...], preferred_element_type=jnp.float32)
    deg = jnp.sum(cnt_ref[...], axis=1, keepdims=True) + 1.0
    dinv = lax.rsqrt(deg)
    h_ref[...] = h
    g_ref[...] = h * dinv


def _mm1(x, w1, cnt_t, nb):
    n, d = x.shape
    h = w1.shape[1]
    grid = n // nb
    return pl.pallas_call(
        _mm1_body,
        grid=(grid,),
        in_specs=[
            pl.BlockSpec((nb, d), lambda i: (i, 0)),
            pl.BlockSpec((d, h), lambda i: (0, 0)),
            pl.BlockSpec((nb, NW), lambda i: (i, 0)),
        ],
        out_specs=[
            pl.BlockSpec((nb, h), lambda i: (i, 0)),
            pl.BlockSpec((nb, h), lambda i: (i, 0)),
        ],
        out_shape=[
            jax.ShapeDtypeStruct((n, h), jnp.float32),
            jax.ShapeDtypeStruct((n, h), jnp.float32),
        ],
    )(x, w1, cnt_t)


def _agg1(g1, src2, dst2, zeros, n_pad):
    h = g1.shape[1]
    nch, c = src2.shape[1], src2.shape[2]
    rpt = n_pad // NS

    @functools.partial(
        pl.kernel,
        out_type=jax.ShapeDtypeStruct((NC, n_pad, h), jnp.float32),
        mesh=_MESH,
        compiler_params=_SC_PARAMS,
        scratch_types=[
            pltpu.VMEM((nch, c), jnp.int32),
            pltpu.VMEM((nch, c), jnp.int32),
            pltpu.VMEM((c, h), jnp.float32),
            pltpu.VMEM_SHARED((n_pad, h), jnp.float32),
            pltpu.SemaphoreType.DMA,
        ],
    )
    def agg_kernel(g_hbm, src_hbm, dst_hbm, z_hbm, out_hbm,
                   src_v, dst_v, rows_v, acc_sh, sem):
        cc = lax.axis_index("c")
        s = lax.axis_index("s")
        wid = cc * NS + s
        off = s * rpt
        pltpu.sync_copy(z_hbm.at[pl.ds(off, rpt)], acc_sh.at[pl.ds(off, rpt)])
        pltpu.sync_copy(src_hbm.at[wid], src_v)
        pltpu.sync_copy(dst_hbm.at[wid], dst_v)
        plsc.subcore_barrier()

        def body(i, _):
            pltpu.async_copy(g_hbm.at[src_v.at[i]], rows_v, sem).wait()
            pltpu.sync_copy(rows_v, acc_sh.at[dst_v.at[i]], add=True)
            return 0

        lax.fori_loop(0, nch, body, 0)
        plsc.subcore_barrier()
        pltpu.sync_copy(acc_sh.at[pl.ds(off, rpt)],
                        out_hbm.at[cc, pl.ds(off, rpt)])

    return agg_kernel(g1, src2, dst2, zeros)


def _mm2_body(sp_ref, h1_ref, cnt_ref, wd_ref, b1_ref, h2d_ref, g2d_ref):
    deg = jnp.sum(cnt_ref[...], axis=1, keepdims=True) + 1.0
    dinv = lax.rsqrt(deg)
    s = sp_ref[0] + sp_ref[1]
    pre = s * dinv + h1_ref[...] * (dinv * dinv) + b1_ref[...]
    u = jnp.maximum(pre, 0.0)
    h2d = jnp.sum(u * wd_ref[...], axis=1, keepdims=True)
    h2d_ref[...] = h2d
    g2d_ref[...] = h2d * dinv


def _mm2(sp, h1, cnt_t, wd_row, b1_row, nb):
    n, h = h1.shape
    grid = n // nb
    return pl.pallas_call(
        _mm2_body,
        grid=(grid,),
        in_specs=[
            pl.BlockSpec((NC, nb, h), lambda i: (0, i, 0)),
            pl.BlockSpec((nb, h), lambda i: (i, 0)),
            pl.BlockSpec((nb, NW), lambda i: (i, 0)),
            pl.BlockSpec((1, h), lambda i: (0, 0)),
            pl.BlockSpec((1, h), lambda i: (0, 0)),
        ],
        out_specs=[
            pl.BlockSpec((nb, 1), lambda i: (i, 0)),
            pl.BlockSpec((nb, 1), lambda i: (i, 0)),
        ],
        out_shape=[
            jax.ShapeDtypeStruct((n, 1), jnp.float32),
            jax.ShapeDtypeStruct((n, 1), jnp.float32),
        ],
    )(sp, h1, cnt_t, wd_row, b1_row)


def _agg2(g2d, src, dst, n_nodes):
    e = src.shape[0]
    epw = e // NW

    @functools.partial(
        pl.kernel,
        out_type=jax.ShapeDtypeStruct((NW, n_nodes), jnp.float32),
        mesh=_MESH,
        compiler_params=_SC_PARAMS,
        scratch_types=[
            pltpu.VMEM((n_nodes,), jnp.float32),
            pltpu.VMEM((n_nodes,), jnp.float32),
            pltpu.VMEM((epw,), jnp.int32),
            pltpu.VMEM((epw,), jnp.int32),
        ],
    )
    def agg2_kernel(g_hbm, src_hbm, dst_hbm, out_hbm,
                    tab_v, acc_v, src_v, dst_v):
        c = lax.axis_index("c")
        s = lax.axis_index("s")
        wid = c * NS + s
        base = wid * epw
        pltpu.sync_copy(g_hbm, tab_v)
        pltpu.sync_copy(src_hbm.at[pl.ds(base, epw)], src_v)
        pltpu.sync_copy(dst_hbm.at[pl.ds(base, epw)], dst_v)

        def zero_body(i, _):
            acc_v[pl.ds(i * L, L)] = jnp.zeros((L,), jnp.float32)
            return 0

        lax.fori_loop(0, n_nodes // L, zero_body, 0)

        def body(i, _):
            sv = src_v[pl.ds(i * L, L)]
            dv = dst_v[pl.ds(i * L, L)]
            vals = plsc.load_gather(tab_v, [sv])
            plsc.addupdate_scatter(acc_v, [dv], vals)
            return 0

        lax.fori_loop(0, epw // L, body, 0)
        pltpu.sync_copy(acc_v, out_hbm.at[wid])

    return agg2_kernel(g2d, src, dst)


def _fin_body(s2_ref, h2d_ref, cnt_ref, bd_ref, p_ref):
    deg = jnp.sum(cnt_ref[...], axis=1, keepdims=True) + 1.0
    dinv = lax.rsqrt(deg)
    s2 = jnp.sum(s2_ref[...], axis=1, keepdims=True)
    dlt = dinv * (s2 + dinv * h2d_ref[...]) + bd_ref[...]
    p0 = 1.0 / (1.0 + jnp.exp(-dlt))
    p1 = 1.0 / (1.0 + jnp.exp(dlt))
    p_ref[...] = jnp.concatenate([p0, p1], axis=1)


def _final(s2_t, h2d, cnt_t, bd, nb):
    n = h2d.shape[0]
    grid = n // nb
    return pl.pallas_call(
        _fin_body,
        grid=(grid,),
        in_specs=[
            pl.BlockSpec((nb, NW), lambda i: (i, 0)),
            pl.BlockSpec((nb, 1), lambda i: (i, 0)),
            pl.BlockSpec((nb, NW), lambda i: (i, 0)),
            pl.BlockSpec((1, 1), lambda i: (0, 0)),
        ],
        out_specs=pl.BlockSpec((nb, 2), lambda i: (i, 0)),
        out_shape=jax.ShapeDtypeStruct((n, 2), jnp.float32),
    )(s2_t, h2d, cnt_t, bd)


def kernel(x, edge_index, W1, b1, W2, b2):
    n, d = x.shape
    h = W1.shape[1]
    e = edge_index.shape[1]
    assert e % NW == 0 and n % NS == 0 and n % L == 0

    epw = e // NW
    chunk = 100
    assert epw % chunk == 0
    nch = epw // chunk
    nb = 2000
    assert n % nb == 0 and nb % 8 == 0

    src = edge_index[0]
    dst = edge_index[1]
    n_pad = 10240
    assert n_pad % (NS * 8) == 0 and n_pad >= n

    src2 = src.reshape(NW, nch, chunk)
    dst2 = dst.reshape(NW, nch, chunk)
    zeros = jnp.zeros((n_pad, h), jnp.float32)

    counts = _degree(dst, n)
    cnt_t = counts.T
    h1, g1 = _mm1(x, W1, cnt_t, nb)
    s1p = _agg1(g1, src2, dst2, zeros, n_pad)
    wd_row = (W2[:, 0] - W2[:, 1]).reshape(1, h)
    b1_row = b1.reshape(1, h)
    h2d, g2d = _mm2(s1p, h1, cnt_t, wd_row, b1_row, nb)
    s2p = _agg2(g2d[:, 0], src, dst, n)
    bd = (b2[0] - b2[1]).reshape(1, 1)
    return _final(s2p.T, h2d, cnt_t, bd, nb)


# speedup vs baseline: 43.8876x; 1.3471x over previous
import functools

import jax
import jax.numpy as jnp
from jax import lax
from jax.experimental import pallas as pl
from jax.experimental.pallas import tpu as pltpu
from jax.experimental.pallas import tpu_sc as plsc

NC = 2
NS = 16
NW = NC * NS
L = 16

_MESH = plsc.VectorSubcoreMesh(core_axis_name="c", subcore_axis_name="s")
_SC_PARAMS = pltpu.CompilerParams(needs_layout_passes=False)


def _degree(dst, n_nodes):
    e = dst.shape[0]
    epw = e // NW

    @functools.partial(
        pl.kernel,
        out_type=jax.ShapeDtypeStruct((NW, n_nodes), jnp.float32),
        mesh=_MESH,
        compiler_params=_SC_PARAMS,
        scratch_types=[
            pltpu.VMEM((n_nodes,), jnp.float32),
            pltpu.VMEM((epw,), jnp.int32),
        ],
    )
    def deg_kernel(dst_hbm, out_hbm, cnt_v, idx_v):
        c = lax.axis_index("c")
        s = lax.axis_index("s")
        wid = c * NS + s
        pltpu.sync_copy(dst_hbm.at[pl.ds(wid * epw, epw)], idx_v)

        def zero_body(i, _):
            cnt_v[pl.ds(i * L, L)] = jnp.zeros((L,), jnp.float32)
            return 0

        lax.fori_loop(0, n_nodes // L, zero_body, 0)

        ones = jnp.full((L,), 1.0, jnp.float32)

        def body(i, _):
            idx = idx_v[pl.ds(i * L, L)]
            plsc.addupdate_scatter(cnt_v, [idx], ones)
            return 0

        lax.fori_loop(0, epw // L, body, 0)
        pltpu.sync_copy(cnt_v, out_hbm.at[wid])

    return deg_kernel(dst)


def _mm1_body(x_ref, w_ref, cnt_ref, h_ref, g_ref):
    h = jnp.dot(x_ref[...], w_ref[...], preferred_element_type=jnp.float32)
    deg = jnp.sum(cnt_ref[...], axis=1, keepdims=True) + 1.0
    dinv = lax.rsqrt(deg)
    h_ref[...] = h
    g_ref[...] = h * dinv


def _mm1(x, w1, cnt_t, nb):
    n, d = x.shape
    h = w1.shape[1]
    grid = n // nb
    return pl.pallas_call(
        _mm1_body,
        grid=(grid,),
        in_specs=[
            pl.BlockSpec((nb, d), lambda i: (i, 0)),
            pl.BlockSpec((d, h), lambda i: (0, 0)),
            pl.BlockSpec((nb, NW), lambda i: (i, 0)),
        ],
        out_specs=[
            pl.BlockSpec((nb, h), lambda i: (i, 0)),
            pl.BlockSpec((nb, h), lambda i: (i, 0)),
        ],
        out_shape=[
            jax.ShapeDtypeStruct((n, h), jnp.float32),
            jax.ShapeDtypeStruct((n, h), jnp.float32),
        ],
    )(x, w1, cnt_t)


def _agg1(g1, src3, dst2, n_pad):
    h = g1.shape[1]
    nch, c = dst2.shape[1], dst2.shape[2]
    rpt = n_pad // NS
    assert nch % 2 == 0 and rpt % 8 == 0

    @functools.partial(
        pl.kernel,
        out_type=jax.ShapeDtypeStruct((NC, n_pad, h), jnp.float32),
        mesh=_MESH,
        compiler_params=_SC_PARAMS,
        scratch_types=[
            pltpu.VMEM((c,), jnp.int32),
            pltpu.VMEM((c,), jnp.int32),
            pltpu.VMEM((nch, c), jnp.int32),
            pltpu.VMEM((c, h), jnp.float32),
            pltpu.VMEM((c, h), jnp.float32),
            pltpu.VMEM_SHARED((n_pad, h), jnp.float32),
            pltpu.SemaphoreType.DMA,
            pltpu.SemaphoreType.DMA,
            pltpu.SemaphoreType.DMA,
            pltpu.SemaphoreType.DMA,
            pltpu.SemaphoreType.DMA,
            pltpu.SemaphoreType.DMA,
        ],
    )
    def agg_kernel(g_hbm, src_hbm, dst_hbm, out_hbm,
                   srcc0, srcc1, dst_v, rows0_v, rows1_v, acc_sh,
                   sg0, sg1, ss0, ss1, si0, si1):
        cc = lax.axis_index("c")
        s = lax.axis_index("s")
        wid = cc * NS + s
        off = s * rpt

        def zbody(i, _):
            r = i // (h // L)
            cs = (i % (h // L)) * L
            rows0_v[r, pl.ds(cs, L)] = jnp.zeros((L,), jnp.float32)
            return 0

        lax.fori_loop(0, c * h // L, zbody, 0)
        for k in range(rpt // c):
            pltpu.sync_copy(rows0_v, acc_sh.at[pl.ds(off + k * c, c)])
        rem = rpt % c
        if rem:
            pltpu.sync_copy(rows0_v.at[pl.ds(0, rem)],
                            acc_sh.at[pl.ds(off + (rpt // c) * c, rem)])
        pltpu.sync_copy(dst_hbm.at[wid], dst_v)
        plsc.subcore_barrier()

        srcc = (srcc0, srcc1)
        rows = (rows0_v, rows1_v)
        sgs = (sg0, sg1)
        sss = (ss0, ss1)
        sis = (si0, si1)
        row0 = wid * nch

        def idx_copy(i, b):
            return pltpu.async_copy(src_hbm.at[row0 + i], srcc[b], sis[b])

        def gather(b):
            return pltpu.async_copy(g_hbm.at[srcc[b]], rows[b], sgs[b])

        def scatter(i, b):
            return pltpu.async_copy(rows[b], acc_sh.at[dst_v.at[i]],
                                    sss[b], add=True)

        idx_copy(0, 0).wait()
        idx_copy(1, 1).wait()
        g0 = gather(0)
        g1_ = gather(1)

        def pipe_body(i, _):
            j = 2 * i
            g0.wait()
            is0 = idx_copy(j + 2, 0)
            sc0 = scatter(j, 0)
            g1_.wait()
            is1 = idx_copy(j + 3, 1)
            sc0.wait()
            is0.wait()
            gather(0)
            sc1 = scatter(j + 1, 1)
            sc1.wait()
            is1.wait()
            gather(1)
            return 0

        lax.fori_loop(0, nch // 2 - 1, pipe_body, 0)
        g0.wait()
        scatter(nch - 2, 0).wait()
        g1_.wait()
        scatter(nch - 1, 1).wait()

        plsc.subcore_barrier()
        pltpu.sync_copy(acc_sh.at[pl.ds(off, rpt)],
                        out_hbm.at[cc, pl.ds(off, rpt)])

    return agg_kernel(g1, src3, dst2)


def _mm2_body(sp_ref, h1_ref, cnt_ref, wd_ref, b1_ref, h2d_ref, g2d_ref):
    deg = jnp.sum(cnt_ref[...], axis=1, keepdims=True) + 1.0
    dinv = lax.rsqrt(deg)
    s = sp_ref[0] + sp_ref[1]
    pre = s * dinv + h1_ref[...] * (dinv * dinv) + b1_ref[...]
    u = jnp.maximum(pre, 0.0)
    h2d = jnp.sum(u * wd_ref[...], axis=1, keepdims=True)
    h2d_ref[...] = h2d
    g2d_ref[...] = h2d * dinv


def _mm2(sp, h1, cnt_t, wd_row, b1_row, nb):
    n, h = h1.shape
    grid = n // nb
    return pl.pallas_call(
        _mm2_body,
        grid=(grid,),
        in_specs=[
            pl.BlockSpec((NC, nb, h), lambda i: (0, i, 0)),
            pl.BlockSpec((nb, h), lambda i: (i, 0)),
            pl.BlockSpec((nb, NW), lambda i: (i, 0)),
            pl.BlockSpec((1, h), lambda i: (0, 0)),
            pl.BlockSpec((1, h), lambda i: (0, 0)),
        ],
        out_specs=[
            pl.BlockSpec((nb, 1), lambda i: (i, 0)),
            pl.BlockSpec((nb, 1), lambda i: (i, 0)),
        ],
        out_shape=[
            jax.ShapeDtypeStruct((n, 1), jnp.float32),
            jax.ShapeDtypeStruct((n, 1), jnp.float32),
        ],
    )(sp, h1, cnt_t, wd_row, b1_row)


def _agg2(g2d, src, dst, n_nodes):
    e = src.shape[0]
    epw = e // NW

    @functools.partial(
        pl.kernel,
        out_type=jax.ShapeDtypeStruct((NW, n_nodes), jnp.float32),
        mesh=_MESH,
        compiler_params=_SC_PARAMS,
        scratch_types=[
            pltpu.VMEM((n_nodes,), jnp.float32),
            pltpu.VMEM((n_nodes,), jnp.float32),
            pltpu.VMEM((epw,), jnp.int32),
            pltpu.VMEM((epw,), jnp.int32),
        ],
    )
    def agg2_kernel(g_hbm, src_hbm, dst_hbm, out_hbm,
                    tab_v, acc_v, src_v, dst_v):
        c = lax.axis_index("c")
        s = lax.axis_index("s")
        wid = c * NS + s
        base = wid * epw
        pltpu.sync_copy(g_hbm, tab_v)
        pltpu.sync_copy(src_hbm.at[pl.ds(base, epw)], src_v)
        pltpu.sync_copy(dst_hbm.at[pl.ds(base, epw)], dst_v)

        def zero_body(i, _):
            acc_v[pl.ds(i * L, L)] = jnp.zeros((L,), jnp.float32)
            return 0

        lax.fori_loop(0, n_nodes // L, zero_body, 0)

        def body(i, _):
            sv = src_v[pl.ds(i * L, L)]
            dv = dst_v[pl.ds(i * L, L)]
            vals = plsc.load_gather(tab_v, [sv])
            plsc.addupdate_scatter(acc_v, [dv], vals)
            return 0

        lax.fori_loop(0, epw // L, body, 0)
        pltpu.sync_copy(acc_v, out_hbm.at[wid])

    return agg2_kernel(g2d, src, dst)


def _fin_body(s2_ref, h2d_ref, cnt_ref, bd_ref, p_ref):
    deg = jnp.sum(cnt_ref[...], axis=1, keepdims=True) + 1.0
    dinv = lax.rsqrt(deg)
    s2 = jnp.sum(s2_ref[...], axis=1, keepdims=True)
    dlt = dinv * (s2 + dinv * h2d_ref[...]) + bd_ref[...]
    p0 = 1.0 / (1.0 + jnp.exp(-dlt))
    p1 = 1.0 / (1.0 + jnp.exp(dlt))
    p_ref[...] = jnp.concatenate([p0, p1], axis=1)


def _final(s2_t, h2d, cnt_t, bd, nb):
    n = h2d.shape[0]
    grid = n // nb
    return pl.pallas_call(
        _fin_body,
        grid=(grid,),
        in_specs=[
            pl.BlockSpec((nb, NW), lambda i: (i, 0)),
            pl.BlockSpec((nb, 1), lambda i: (i, 0)),
            pl.BlockSpec((nb, NW), lambda i: (i, 0)),
            pl.BlockSpec((1, 1), lambda i: (0, 0)),
        ],
        out_specs=pl.BlockSpec((nb, 2), lambda i: (i, 0)),
        out_shape=jax.ShapeDtypeStruct((n, 2), jnp.float32),
    )(s2_t, h2d, cnt_t, bd)


def kernel(x, edge_index, W1, b1, W2, b2):
    n, d = x.shape
    h = W1.shape[1]
    e = edge_index.shape[1]
    assert e % NW == 0 and n % NS == 0 and n % L == 0

    epw = e // NW
    chunk = 100
    assert epw % chunk == 0
    nch = epw // chunk
    nb = 2000
    assert n % nb == 0 and nb % 8 == 0

    src = edge_index[0]
    dst = edge_index[1]
    n_pad = 10240
    assert n_pad % (NS * 8) == 0 and n_pad >= n

    src3 = src.reshape(NW * nch, chunk)
    dst2 = dst.reshape(NW, nch, chunk)

    counts = _degree(dst, n)
    cnt_t = counts.T
    h1, g1 = _mm1(x, W1, cnt_t, nb)
    s1p = _agg1(g1, src3, dst2, n_pad)
    wd_row = (W2[:, 0] - W2[:, 1]).reshape(1, h)
    b1_row = b1.reshape(1, h)
    h2d, g2d = _mm2(s1p, h1, cnt_t, wd_row, b1_row, nb)
    s2p = _agg2(g2d[:, 0], src, dst, n)
    bd = (b2[0] - b2[1]).reshape(1, 1)
    return _final(s2p.T, h2d, cnt_t, bd, nb)


# speedup vs baseline: 45.2843x; 1.0318x over previous
import functools

import jax
import jax.numpy as jnp
from jax import lax
from jax.experimental import pallas as pl
from jax.experimental.pallas import tpu as pltpu
from jax.experimental.pallas import tpu_sc as plsc

NC = 2
NS = 16
NW = NC * NS
L = 16

_MESH = plsc.VectorSubcoreMesh(core_axis_name="c", subcore_axis_name="s")
_SC_PARAMS = pltpu.CompilerParams(needs_layout_passes=False)


def _degree(dst, n_nodes):
    e = dst.shape[0]
    epw = e // NW

    @functools.partial(
        pl.kernel,
        out_type=jax.ShapeDtypeStruct((NW, n_nodes), jnp.float32),
        mesh=_MESH,
        compiler_params=_SC_PARAMS,
        scratch_types=[
            pltpu.VMEM((n_nodes,), jnp.float32),
            pltpu.VMEM((epw,), jnp.int32),
        ],
    )
    def deg_kernel(dst_hbm, out_hbm, cnt_v, idx_v):
        c = lax.axis_index("c")
        s = lax.axis_index("s")
        wid = c * NS + s
        pltpu.sync_copy(dst_hbm.at[pl.ds(wid * epw, epw)], idx_v)

        def zero_body(i, _):
            cnt_v[pl.ds(i * L, L)] = jnp.zeros((L,), jnp.float32)
            return 0

        lax.fori_loop(0, n_nodes // L, zero_body, 0)

        ones = jnp.full((L,), 1.0, jnp.float32)

        def body(i, _):
            idx = idx_v[pl.ds(i * L, L)]
            plsc.addupdate_scatter(cnt_v, [idx], ones)
            return 0

        lax.fori_loop(0, epw // L, body, 0)
        pltpu.sync_copy(cnt_v, out_hbm.at[wid])

    return deg_kernel(dst)


def _mm1_body(x_ref, w_ref, cnt_ref, h_ref, g_ref):
    h = jnp.dot(x_ref[...], w_ref[...], preferred_element_type=jnp.float32)
    deg = jnp.sum(cnt_ref[...], axis=1, keepdims=True) + 1.0
    dinv = lax.rsqrt(deg)
    h_ref[...] = h
    g_ref[...] = h * dinv


def _mm1(x, w1, cnt_t, nb):
    n, d = x.shape
    h = w1.shape[1]
    grid = n // nb
    return pl.pallas_call(
        _mm1_body,
        grid=(grid,),
        in_specs=[
            pl.BlockSpec((nb, d), lambda i: (i, 0)),
            pl.BlockSpec((d, h), lambda i: (0, 0)),
            pl.BlockSpec((nb, NW), lambda i: (i, 0)),
        ],
        out_specs=[
            pl.BlockSpec((nb, h), lambda i: (i, 0)),
            pl.BlockSpec((nb, h), lambda i: (i, 0)),
        ],
        out_shape=[
            jax.ShapeDtypeStruct((n, h), jnp.float32),
            jax.ShapeDtypeStruct((n, h), jnp.float32),
        ],
    )(x, w1, cnt_t)


def _agg1(g1, src_p, dst_p, n_pad, c):
    h = g1.shape[1]
    ep = src_p.shape[0]
    epw = ep // NW
    nch = epw // c
    rpt = n_pad // NS
    assert nch % 4 == 0 and rpt % c == 0 and epw % c == 0

    @functools.partial(
        pl.kernel,
        out_type=jax.ShapeDtypeStruct((NC, n_pad, h), jnp.float32),
        mesh=_MESH,
        compiler_params=_SC_PARAMS,
        scratch_types=[
            [pltpu.VMEM((c,), jnp.int32) for _ in range(4)],
            [pltpu.VMEM((c,), jnp.int32) for _ in range(4)],
            [pltpu.VMEM((c, h), jnp.float32) for _ in range(2)],
            pltpu.VMEM_SHARED((n_pad, h), jnp.float32),
            [pltpu.SemaphoreType.DMA for _ in range(4)],
            [pltpu.SemaphoreType.DMA for _ in range(2)],
            [pltpu.SemaphoreType.DMA for _ in range(2)],
        ],
    )
    def agg_kernel(g_hbm, src_hbm, dst_hbm, out_hbm,
                   srcc, dstc, rows, acc_sh, si, sg, ss):
        cc = lax.axis_index("c")
        s = lax.axis_index("s")
        wid = cc * NS + s
        off = s * rpt
        base = wid * epw

        def zbody(i, _):
            r = i // (h // L)
            cs = (i % (h // L)) * L
            rows[0][r, pl.ds(cs, L)] = jnp.zeros((L,), jnp.float32)
            return 0

        lax.fori_loop(0, c * h // L, zbody, 0)
        for k in range(rpt // c):
            pltpu.sync_copy(rows[0], acc_sh.at[pl.ds(off + k * c, c)])
        plsc.subcore_barrier()

        def idx_start(i, q):
            pltpu.async_copy(src_hbm.at[pl.ds(base + i * c, c)], srcc[q],
                             si[q])
            pltpu.async_copy(dst_hbm.at[pl.ds(base + i * c, c)], dstc[q],
                             si[q])

        def idx_wait(q):
            pltpu.make_async_copy(src_hbm.at[pl.ds(0, c)], srcc[q],
                                  si[q]).wait()
            pltpu.make_async_copy(dst_hbm.at[pl.ds(0, c)], dstc[q],
                                  si[q]).wait()

        def gather_start(q, r):
            pltpu.async_copy(g_hbm.at[srcc[q]], rows[r], sg[r])

        def gather_wait(r):
            pltpu.make_async_copy(g_hbm.at[srcc[0]], rows[r], sg[r]).wait()

        def scatter_start(q, r):
            pltpu.async_copy(rows[r], acc_sh.at[dstc[q]], ss[r], add=True)

        def scatter_wait(r):
            pltpu.make_async_copy(rows[r], acc_sh.at[dstc[0]], ss[r]).wait()

        for q in range(4):
            idx_start(q, q)
        idx_wait(0)
        gather_start(0, 0)
        idx_wait(1)
        gather_start(1, 1)

        def quad(i, _):
            j = 4 * i
            for t in range(4):
                q = t
                r = t % 2
                gather_wait(r)
                scatter_start(q, r)
                scatter_wait(r)
                idx_start(j + 4 + t, q)
                if t >= 2:
                    idx_wait((t + 2) % 4)
                    gather_start((t + 2) % 4, r)
                else:
                    idx_wait(t + 2)
                    gather_start(t + 2, r)
            return 0

        lax.fori_loop(0, nch // 4 - 1, quad, 0)

        for t in range(4):
            r = t % 2
            gather_wait(r)
            scatter_start(t, r)
            scatter_wait(r)
            if t < 2:
                idx_wait(t + 2)
                gather_start(t + 2, r)

        plsc.subcore_barrier()
        pltpu.sync_copy(acc_sh.at[pl.ds(off, rpt)],
                        out_hbm.at[cc, pl.ds(off, rpt)])

    return agg_kernel(g1, src_p, dst_p)


def _mm2_body(sp_ref, h1_ref, cnt_ref, wd_ref, b1_ref, h2d_ref, g2d_ref):
    deg = jnp.sum(cnt_ref[...], axis=1, keepdims=True) + 1.0
    dinv = lax.rsqrt(deg)
    s = sp_ref[0] + sp_ref[1]
    pre = s * dinv + h1_ref[...] * (dinv * dinv) + b1_ref[...]
    u = jnp.maximum(pre, 0.0)
    h2d = jnp.sum(u * wd_ref[...], axis=1, keepdims=True)
    h2d_ref[...] = h2d
    g2d_ref[...] = h2d * dinv


def _mm2(sp, h1, cnt_t, wd_row, b1_row, nb):
    n, h = h1.shape
    grid = n // nb
    return pl.pallas_call(
        _mm2_body,
        grid=(grid,),
        in_specs=[
            pl.BlockSpec((NC, nb, h), lambda i: (0, i, 0)),
            pl.BlockSpec((nb, h), lambda i: (i, 0)),
            pl.BlockSpec((nb, NW), lambda i: (i, 0)),
            pl.BlockSpec((1, h), lambda i: (0, 0)),
            pl.BlockSpec((1, h), lambda i: (0, 0)),
        ],
        out_specs=[
            pl.BlockSpec((nb, 1), lambda i: (i, 0)),
            pl.BlockSpec((nb, 1), lambda i: (i, 0)),
        ],
        out_shape=[
            jax.ShapeDtypeStruct((n, 1), jnp.float32),
            jax.ShapeDtypeStruct((n, 1), jnp.float32),
        ],
    )(sp, h1, cnt_t, wd_row, b1_row)


def _agg2(g2d, src, dst, n_nodes):
    e = src.shape[0]
    epw = e // NW

    @functools.partial(
        pl.kernel,
        out_type=jax.ShapeDtypeStruct((NW, n_nodes), jnp.float32),
        mesh=_MESH,
        compiler_params=_SC_PARAMS,
        scratch_types=[
            pltpu.VMEM((n_nodes,), jnp.float32),
            pltpu.VMEM((n_nodes,), jnp.float32),
            pltpu.VMEM((epw,), jnp.int32),
            pltpu.VMEM((epw,), jnp.int32),
        ],
    )
    def agg2_kernel(g_hbm, src_hbm, dst_hbm, out_hbm,
                    tab_v, acc_v, src_v, dst_v):
        c = lax.axis_index("c")
        s = lax.axis_index("s")
        wid = c * NS + s
        base = wid * epw
        pltpu.sync_copy(g_hbm, tab_v)
        pltpu.sync_copy(src_hbm.at[pl.ds(base, epw)], src_v)
        pltpu.sync_copy(dst_hbm.at[pl.ds(base, epw)], dst_v)

        def zero_body(i, _):
            acc_v[pl.ds(i * L, L)] = jnp.zeros((L,), jnp.float32)
            return 0

        lax.fori_loop(0, n_nodes // L, zero_body, 0)

        def body(i, _):
            sv = src_v[pl.ds(i * L, L)]
            dv = dst_v[pl.ds(i * L, L)]
            vals = plsc.load_gather(tab_v, [sv])
            plsc.addupdate_scatter(acc_v, [dv], vals)
            return 0

        lax.fori_loop(0, epw // L, body, 0)
        pltpu.sync_copy(acc_v, out_hbm.at[wid])

    return agg2_kernel(g2d, src, dst)


def _fin_body(s2_ref, h2d_ref, cnt_ref, bd_ref, p_ref):
    deg = jnp.sum(cnt_ref[...], axis=1, keepdims=True) + 1.0
    dinv = lax.rsqrt(deg)
    s2 = jnp.sum(s2_ref[...], axis=1, keepdims=True)
    dlt = dinv * (s2 + dinv * h2d_ref[...]) + bd_ref[...]
    p0 = 1.0 / (1.0 + jnp.exp(-dlt))
    p1 = 1.0 / (1.0 + jnp.exp(dlt))
    p_ref[...] = jnp.concatenate([p0, p1], axis=1)


def _final(s2_t, h2d, cnt_t, bd, nb):
    n = h2d.shape[0]
    grid = n // nb
    return pl.pallas_call(
        _fin_body,
        grid=(grid,),
        in_specs=[
            pl.BlockSpec((nb, NW), lambda i: (i, 0)),
            pl.BlockSpec((nb, 1), lambda i: (i, 0)),
            pl.BlockSpec((nb, NW), lambda i: (i, 0)),
            pl.BlockSpec((1, 1), lambda i: (0, 0)),
        ],
        out_specs=pl.BlockSpec((nb, 2), lambda i: (i, 0)),
        out_shape=jax.ShapeDtypeStruct((n, 2), jnp.float32),
    )(s2_t, h2d, cnt_t, bd)


def kernel(x, edge_index, W1, b1, W2, b2):
    n, d = x.shape
    h = W1.shape[1]
    e = edge_index.shape[1]
    assert e % NW == 0 and n % NS == 0 and n % L == 0

    nb = 2000
    assert n % nb == 0 and nb % 8 == 0

    src = edge_index[0]
    dst = edge_index[1]
    n_pad = 10240
    assert n_pad % (NS * 8) == 0 and n_pad > n

    chunk = 128
    epw_p = -(-e // (NW * chunk * 4)) * chunk * 4
    ep = epw_p * NW
    npad_e = ep - e
    pad_src = (jnp.arange(npad_e, dtype=jnp.int32)) % n
    pad_dst = n + (jnp.arange(npad_e, dtype=jnp.int32)) % (n_pad - n)
    src_p = jnp.concatenate([src, pad_src])
    dst_p = jnp.concatenate([dst, pad_dst])

    counts = _degree(dst, n)
    cnt_t = counts.T
    h1, g1 = _mm1(x, W1, cnt_t, nb)
    s1p = _agg1(g1, src_p, dst_p, n_pad, chunk)
    wd_row = (W2[:, 0] - W2[:, 1]).reshape(1, h)
    b1_row = b1.reshape(1, h)
    h2d, g2d = _mm2(s1p, h1, cnt_t, wd_row, b1_row, nb)
    s2p = _agg2(g2d[:, 0], src, dst, n)
    bd = (b2[0] - b2[1]).reshape(1, 1)
    return _final(s2p.T, h2d, cnt_t, bd, nb)
